# Initial kernel scaffold; baseline (speedup 1.0000x reference)
import jax
import jax.numpy as jnp
from jax.experimental import pallas as pl


def kernel(deg_idx, edge_index, batch, emb, Wl0, bl0, Wr0, Wl1, bl1, Wr1):
    raise NotImplementedError("write your pallas kernel here")


# speedup vs baseline: 2.6819x; 2.6819x over previous
import jax
import jax.numpy as jnp
from jax import lax
from jax.experimental import pallas as pl
from jax.experimental.pallas import tpu as pltpu
from jax.experimental.pallas import tpu_sc as plsc

N = 10000
E = 320000
D = 128
NB = 65
NCL = 50
NPC = N // NCL

NC = 2
NS = 16
NW = NC * NS
K = 128
NP = 10240
RPT = NP // NS
EP = NW * NP
EPW = NP
NCHUNK = EPW // K


def _prep_body(di_ref, emb_ref, wl_ref, wr_ref, y_ref, z_ref):
    cols = lax.broadcasted_iota(jnp.int32, (1, NB), 1)
    p = jnp.where(di_ref[...] == cols, 1.0, 0.0).astype(jnp.float32)
    el = jnp.dot(emb_ref[...], wl_ref[...], preferred_element_type=jnp.float32)
    er = jnp.dot(emb_ref[...], wr_ref[...], preferred_element_type=jnp.float32)
    y_ref[...] = jnp.dot(p, el, preferred_element_type=jnp.float32)
    z_ref[...] = jnp.dot(p, er, preferred_element_type=jnp.float32)


def _mid_body(sp_ref, dp_ref, z0_ref, bl0_ref, wl1_ref, wr1_ref,
              y1_ref, z1_ref, dinv_ref):
    s1 = sp_ref[:N, :] + sp_ref[NP:NP + N, :]
    deg = dp_ref[:N, :1] + dp_ref[NP:NP + N, :1]
    dinv = 1.0 / jnp.maximum(deg, 1.0)
    h1 = jnp.maximum(s1 * dinv + bl0_ref[...] + z0_ref[...], 0.0)
    y1_ref[...] = jnp.dot(h1, wl1_ref[...], preferred_element_type=jnp.float32)
    z1_ref[...] = jnp.dot(h1, wr1_ref[...], preferred_element_type=jnp.float32)
    dinv_ref[...] = dinv


def _fin_body(sp_ref, dinv_ref, z1_ref, bl1_ref, out_ref):
    s2 = sp_ref[:N, :] + sp_ref[NP:NP + N, :]
    h2 = jnp.maximum(s2 * dinv_ref[...] + bl1_ref[...] + z1_ref[...], 0.0)
    ji = lax.broadcasted_iota(jnp.int32, (NCL, N), 1)
    ci = lax.broadcasted_iota(jnp.int32, (NCL, N), 0)
    m = jnp.where(ji % NCL == ci, 1.0 / NPC, 0.0)
    out_ref[...] = jnp.dot(m, h2, preferred_element_type=jnp.float32)


_prep = pl.pallas_call(
    _prep_body,
    out_shape=[jax.ShapeDtypeStruct((N, D), jnp.float32),
               jax.ShapeDtypeStruct((N, D), jnp.float32)],
)

_mid = pl.pallas_call(
    _mid_body,
    out_shape=[jax.ShapeDtypeStruct((N, D), jnp.float32),
               jax.ShapeDtypeStruct((N, D), jnp.float32),
               jax.ShapeDtypeStruct((N, 1), jnp.float32)],
)

_fin = pl.pallas_call(
    _fin_body,
    out_shape=[jax.ShapeDtypeStruct((NCL, D), jnp.float32)],
)


_SC_MESH = plsc.VectorSubcoreMesh(core_axis_name="c", subcore_axis_name="s")


def _seg_body(y_hbm, src_hbm, dst_hbm, zer_hbm, out_hbm,
              src_v, dst_v, rows_v, acc_sh, sem):
    c = lax.axis_index("c")
    s = lax.axis_index("s")
    w = s * NC + c
    row0 = s * RPT

    pltpu.sync_copy(zer_hbm, acc_sh.at[pl.ds(row0, RPT)])
    plsc.subcore_barrier()

    base = w * EPW

    def chunk(i, carry):
        off = base + i * K
        pltpu.sync_copy(src_hbm.at[pl.ds(off, K)], src_v)
        pltpu.sync_copy(dst_hbm.at[pl.ds(off, K)], dst_v)
        pltpu.async_copy(y_hbm.at[src_v], rows_v, sem).wait()
        pltpu.sync_copy(rows_v, acc_sh.at[dst_v], add=True)
        return carry

    lax.fori_loop(0, NCHUNK, chunk, 0)
    plsc.subcore_barrier()

    pltpu.sync_copy(acc_sh.at[pl.ds(row0, RPT)],
                    out_hbm.at[pl.ds(c * NP + row0, RPT)])


def _degree_body(dst_hbm, ones_hbm, zer_hbm, out_hbm,
                 dst_v, ones_v, acc_sh):
    c = lax.axis_index("c")
    s = lax.axis_index("s")
    w = s * NC + c
    row0 = s * RPT

    pltpu.sync_copy(zer_hbm, acc_sh.at[pl.ds(row0, RPT)])
    pltpu.sync_copy(ones_hbm, ones_v)
    plsc.subcore_barrier()

    base = w * EPW

    def chunk(i, carry):
        off = base + i * K
        pltpu.sync_copy(dst_hbm.at[pl.ds(off, K)], dst_v)
        pltpu.sync_copy(ones_v, acc_sh.at[dst_v], add=True)
        return carry

    lax.fori_loop(0, NCHUNK, chunk, 0)
    plsc.subcore_barrier()

    pltpu.sync_copy(acc_sh.at[pl.ds(row0, RPT)],
                    out_hbm.at[pl.ds(c * NP + row0, RPT)])


_seg = pl.kernel(
    _seg_body,
    out_type=[jax.ShapeDtypeStruct((NC * NP, D), jnp.float32)],
    mesh=_SC_MESH,
    scratch_types=[
        pltpu.VMEM((K,), jnp.int32),
        pltpu.VMEM((K,), jnp.int32),
        pltpu.VMEM((K, D), jnp.float32),
        pltpu.VMEM_SHARED((NP, D), jnp.float32),
        pltpu.SemaphoreType.DMA,
    ],
)

_degree = pl.kernel(
    _degree_body,
    out_type=[jax.ShapeDtypeStruct((NC * NP, D), jnp.float32)],
    mesh=_SC_MESH,
    scratch_types=[
        pltpu.VMEM((K,), jnp.int32),
        pltpu.VMEM((K, D), jnp.float32),
        pltpu.VMEM_SHARED((NP, D), jnp.float32),
    ],
)


@jax.jit
def _run(deg_idx, edge_index, emb, Wl0, bl0, Wr0, Wl1, bl1, Wr1):
    src = jnp.concatenate([edge_index[0], jnp.zeros((EP - E,), jnp.int32)])
    dst = jnp.concatenate([edge_index[1],
                           jnp.full((EP - E,), NP - 1, jnp.int32)])
    di = deg_idx.reshape(N, 1).astype(jnp.int32)

    zeros_a = jnp.zeros((RPT, D), jnp.float32)
    ones_k = jnp.ones((K, D), jnp.float32)

    (dp,) = _degree(dst, ones_k, zeros_a)

    y0, z0 = _prep(di, emb, Wl0, Wr0)

    (s1p,) = _seg(y0, src, dst, zeros_a)

    y1, z1, dinv = _mid(s1p, dp, z0, bl0.reshape(1, D), Wl1, Wr1)

    (s2p,) = _seg(y1, src, dst, zeros_a)

    (pooled,) = _fin(s2p, dinv, z1, bl1.reshape(1, D))
    return pooled[None]


def kernel(deg_idx, edge_index, batch, emb, Wl0, bl0, Wr0, Wl1, bl1, Wr1):
    del batch
    return _run(deg_idx, edge_index, emb, Wl0, bl0, Wr0, Wl1, bl1, Wr1)


# speedup vs baseline: 3.0853x; 1.1504x over previous
import jax
import jax.numpy as jnp
from jax import lax
from jax.experimental import pallas as pl
from jax.experimental.pallas import tpu as pltpu
from jax.experimental.pallas import tpu_sc as plsc

N = 10000
E = 320000
D = 128
NB = 65
NCL = 50
NPC = N // NCL

NC = 2
NS = 16
NW = NC * NS
K = 128
NP = 10240
RPT = NP // NS
EP = NW * NP
EPW = NP
NCHUNK = EPW // K
HC = NCHUNK // 2


def _prep_body(di_ref, emb_ref, wl_ref, wr_ref, y_ref, z_ref):
    cols = lax.broadcasted_iota(jnp.int32, (1, NB), 1)
    p = jnp.where(di_ref[...] == cols, 1.0, 0.0).astype(jnp.float32)
    el = jnp.dot(emb_ref[...], wl_ref[...], preferred_element_type=jnp.float32)
    er = jnp.dot(emb_ref[...], wr_ref[...], preferred_element_type=jnp.float32)
    y_ref[...] = jnp.dot(p, el, preferred_element_type=jnp.float32)
    z_ref[...] = jnp.dot(p, er, preferred_element_type=jnp.float32)


def _mid_body(sp_ref, dp_ref, z0_ref, bl0_ref, wl1_ref, wr1_ref,
              y1_ref, z1_ref, dinv_ref):
    s1 = sp_ref[:N, :] + sp_ref[NP:NP + N, :]
    deg = dp_ref[:N, :1] + dp_ref[NP:NP + N, :1]
    dinv = 1.0 / jnp.maximum(deg, 1.0)
    h1 = jnp.maximum(s1 * dinv + bl0_ref[...] + z0_ref[...], 0.0)
    y1_ref[...] = jnp.dot(h1, wl1_ref[...], preferred_element_type=jnp.float32)
    z1_ref[...] = jnp.dot(h1, wr1_ref[...], preferred_element_type=jnp.float32)
    dinv_ref[...] = dinv


def _fin_body(sp_ref, dinv_ref, z1_ref, bl1_ref, out_ref):
    s2 = sp_ref[:N, :] + sp_ref[NP:NP + N, :]
    h2 = jnp.maximum(s2 * dinv_ref[...] + bl1_ref[...] + z1_ref[...], 0.0)
    ji = lax.broadcasted_iota(jnp.int32, (NCL, N), 1)
    ci = lax.broadcasted_iota(jnp.int32, (NCL, N), 0)
    m = jnp.where(ji % NCL == ci, 1.0 / NPC, 0.0)
    out_ref[...] = jnp.dot(m, h2, preferred_element_type=jnp.float32)


_prep = pl.pallas_call(
    _prep_body,
    out_shape=[jax.ShapeDtypeStruct((N, D), jnp.float32),
               jax.ShapeDtypeStruct((N, D), jnp.float32)],
)

_mid = pl.pallas_call(
    _mid_body,
    out_shape=[jax.ShapeDtypeStruct((N, D), jnp.float32),
               jax.ShapeDtypeStruct((N, D), jnp.float32),
               jax.ShapeDtypeStruct((N, 1), jnp.float32)],
)

_fin = pl.pallas_call(
    _fin_body,
    out_shape=[jax.ShapeDtypeStruct((NCL, D), jnp.float32)],
)


_SC_MESH = plsc.VectorSubcoreMesh(core_axis_name="c", subcore_axis_name="s")


def _seg_body(y_hbm, src_hbm, dst_hbm, zer_hbm, out_hbm,
              src_v, dst_v, rows_a, rows_b, acc_sh, sem_a, sem_b):
    c = lax.axis_index("c")
    s = lax.axis_index("s")
    w = s * NC + c
    row0 = s * RPT

    pltpu.sync_copy(zer_hbm, acc_sh.at[pl.ds(row0, RPT)])
    plsc.subcore_barrier()

    def half(h, carry):
        pltpu.sync_copy(src_hbm.at[pl.ds(w * NCHUNK + h * HC, HC)], src_v)
        pltpu.sync_copy(dst_hbm.at[pl.ds(w * NCHUNK + h * HC, HC)], dst_v)
        pltpu.async_copy(y_hbm.at[src_v.at[0]], rows_a, sem_a)

        def chunk2(j, carry2):
            i0 = 2 * j
            pltpu.async_copy(y_hbm.at[src_v.at[i0 + 1]], rows_b, sem_b)
            pltpu.make_async_copy(y_hbm.at[src_v.at[i0]], rows_a, sem_a).wait()
            pltpu.sync_copy(rows_a, acc_sh.at[dst_v.at[i0]], add=True)

            @pl.when(j < HC // 2 - 1)
            def _():
                pltpu.async_copy(y_hbm.at[src_v.at[i0 + 2]], rows_a, sem_a)

            pltpu.make_async_copy(y_hbm.at[src_v.at[i0 + 1]], rows_b, sem_b).wait()
            pltpu.sync_copy(rows_b, acc_sh.at[dst_v.at[i0 + 1]], add=True)
            return carry2

        lax.fori_loop(0, HC // 2, chunk2, 0)
        return carry

    lax.fori_loop(0, 2, half, 0)
    plsc.subcore_barrier()

    pltpu.sync_copy(acc_sh.at[pl.ds(row0, RPT)],
                    out_hbm.at[pl.ds(c * NP + row0, RPT)])


def _degree_body(dst_hbm, ones_hbm, zer_hbm, out_hbm,
                 dst_v, ones_v, acc_sh):
    c = lax.axis_index("c")
    s = lax.axis_index("s")
    w = s * NC + c
    row0 = s * RPT

    pltpu.sync_copy(zer_hbm, acc_sh.at[pl.ds(row0, RPT)])
    pltpu.sync_copy(ones_hbm, ones_v)
    pltpu.sync_copy(dst_hbm.at[pl.ds(w * NCHUNK, NCHUNK)], dst_v)
    plsc.subcore_barrier()

    def chunk(i, carry):
        pltpu.sync_copy(ones_v, acc_sh.at[dst_v.at[i]], add=True)
        return carry

    lax.fori_loop(0, NCHUNK, chunk, 0)
    plsc.subcore_barrier()

    pltpu.sync_copy(acc_sh.at[pl.ds(row0, RPT)],
                    out_hbm.at[pl.ds(c * NP + row0, RPT)])


_seg = pl.kernel(
    _seg_body,
    out_type=[jax.ShapeDtypeStruct((NC * NP, D), jnp.float32)],
    mesh=_SC_MESH,
    scratch_types=[
        pltpu.VMEM((HC, K), jnp.int32),
        pltpu.VMEM((HC, K), jnp.int32),
        pltpu.VMEM((K, D), jnp.float32),
        pltpu.VMEM((K, D), jnp.float32),
        pltpu.VMEM_SHARED((NP, D), jnp.float32),
        pltpu.SemaphoreType.DMA,
        pltpu.SemaphoreType.DMA,
    ],
)

_degree = pl.kernel(
    _degree_body,
    out_type=[jax.ShapeDtypeStruct((NC * NP, D), jnp.float32)],
    mesh=_SC_MESH,
    scratch_types=[
        pltpu.VMEM((NCHUNK, K), jnp.int32),
        pltpu.VMEM((K, D), jnp.float32),
        pltpu.VMEM_SHARED((NP, D), jnp.float32),
    ],
)


@jax.jit
def _run(deg_idx, edge_index, emb, Wl0, bl0, Wr0, Wl1, bl1, Wr1):
    src = jnp.concatenate(
        [edge_index[0].astype(jnp.int32), jnp.zeros((EP - E,), jnp.int32)]
    ).reshape(NW * NCHUNK, K)
    dst = jnp.concatenate(
        [edge_index[1].astype(jnp.int32),
         jnp.full((EP - E,), NP - 1, jnp.int32)]
    ).reshape(NW * NCHUNK, K)
    di = deg_idx.reshape(N, 1).astype(jnp.int32)

    zeros_a = jnp.zeros((RPT, D), jnp.float32)
    ones_k = jnp.ones((K, D), jnp.float32)

    (dp,) = _degree(dst, ones_k, zeros_a)

    y0, z0 = _prep(di, emb, Wl0, Wr0)

    (s1p,) = _seg(y0, src, dst, zeros_a)

    y1, z1, dinv = _mid(s1p, dp, z0, bl0.reshape(1, D), Wl1, Wr1)

    (s2p,) = _seg(y1, src, dst, zeros_a)

    (pooled,) = _fin(s2p, dinv, z1, bl1.reshape(1, D))
    return pooled[None]


def kernel(deg_idx, edge_index, batch, emb, Wl0, bl0, Wr0, Wl1, bl1, Wr1):
    del batch
    return _run(deg_idx, edge_index, emb, Wl0, bl0, Wr0, Wl1, bl1, Wr1)


# speedup vs baseline: 3.0894x; 1.0013x over previous
import jax
import jax.numpy as jnp
from jax import lax
from jax.experimental import pallas as pl
from jax.experimental.pallas import tpu as pltpu
from jax.experimental.pallas import tpu_sc as plsc

N = 10000
E = 320000
D = 128
NB = 65
NCL = 50
NPC = N // NCL

NC = 2
NS = 16
NW = NC * NS
K = 128
NP = 10240
RPT = NP // NS
EP = NW * NP
EPW = NP
NCHUNK = EPW // K
HC = NCHUNK // 2


def _prep_body(di_ref, emb_ref, wl_ref, wr_ref, y_ref, z_ref):
    cols = lax.broadcasted_iota(jnp.int32, (1, NB), 1)
    p = jnp.where(di_ref[...] == cols, 1.0, 0.0).astype(jnp.float32)
    el = jnp.dot(emb_ref[...], wl_ref[...], preferred_element_type=jnp.float32)
    er = jnp.dot(emb_ref[...], wr_ref[...], preferred_element_type=jnp.float32)
    y_ref[...] = jnp.dot(p, el, preferred_element_type=jnp.float32)
    z_ref[...] = jnp.dot(p, er, preferred_element_type=jnp.float32)


def _mid_body(sp_ref, dp_ref, z0_ref, bl0_ref, wl1_ref, wr1_ref,
              y1_ref, z1_ref, dinv_ref):
    s1 = sp_ref[:N, :] + sp_ref[NP:NP + N, :]
    deg = dp_ref[:N, :1] + dp_ref[NP:NP + N, :1]
    dinv = 1.0 / jnp.maximum(deg, 1.0)
    h1 = jnp.maximum(s1 * dinv + bl0_ref[...] + z0_ref[...], 0.0)
    y1_ref[...] = jnp.dot(h1, wl1_ref[...], preferred_element_type=jnp.float32)
    z1_ref[...] = jnp.dot(h1, wr1_ref[...], preferred_element_type=jnp.float32)
    dinv_ref[...] = dinv


def _fin_body(sp_ref, dinv_ref, z1_ref, bl1_ref, out_ref):
    s2 = sp_ref[:N, :] + sp_ref[NP:NP + N, :]
    h2 = jnp.maximum(s2 * dinv_ref[...] + bl1_ref[...] + z1_ref[...], 0.0)
    ji = lax.broadcasted_iota(jnp.int32, (NCL, N), 1)
    ci = lax.broadcasted_iota(jnp.int32, (NCL, N), 0)
    m = jnp.where(ji % NCL == ci, 1.0 / NPC, 0.0)
    out_ref[...] = jnp.dot(m, h2, preferred_element_type=jnp.float32)


_prep = pl.pallas_call(
    _prep_body,
    out_shape=[jax.ShapeDtypeStruct((N, D), jnp.float32),
               jax.ShapeDtypeStruct((N, D), jnp.float32)],
)

_mid = pl.pallas_call(
    _mid_body,
    out_shape=[jax.ShapeDtypeStruct((N, D), jnp.float32),
               jax.ShapeDtypeStruct((N, D), jnp.float32),
               jax.ShapeDtypeStruct((N, 1), jnp.float32)],
)

_fin = pl.pallas_call(
    _fin_body,
    out_shape=[jax.ShapeDtypeStruct((NCL, D), jnp.float32)],
)


_SC_MESH = plsc.VectorSubcoreMesh(core_axis_name="c", subcore_axis_name="s")


def _seg_body(y_hbm, src_hbm, dst_hbm, zer_hbm, out_hbm,
              src_v, dst_v, rows_a, rows_b, acc_sh, sem_a, sem_b):
    c = lax.axis_index("c")
    s = lax.axis_index("s")
    w = s * NC + c
    row0 = s * RPT

    pltpu.sync_copy(zer_hbm, acc_sh.at[pl.ds(row0, RPT)])
    plsc.subcore_barrier()

    def half(h, carry):
        pltpu.sync_copy(src_hbm.at[pl.ds(w * NCHUNK + h * HC, HC)], src_v)
        pltpu.sync_copy(dst_hbm.at[pl.ds(w * NCHUNK + h * HC, HC)], dst_v)
        pltpu.async_copy(y_hbm.at[src_v.at[0]], rows_a, sem_a)

        def chunk2(j, carry2):
            i0 = 2 * j
            pltpu.async_copy(y_hbm.at[src_v.at[i0 + 1]], rows_b, sem_b)
            pltpu.make_async_copy(y_hbm.at[src_v.at[i0]], rows_a, sem_a).wait()
            pltpu.sync_copy(rows_a, acc_sh.at[dst_v.at[i0]], add=True)

            @pl.when(j < HC // 2 - 1)
            def _():
                pltpu.async_copy(y_hbm.at[src_v.at[i0 + 2]], rows_a, sem_a)

            pltpu.make_async_copy(y_hbm.at[src_v.at[i0 + 1]], rows_b, sem_b).wait()
            pltpu.sync_copy(rows_b, acc_sh.at[dst_v.at[i0 + 1]], add=True)
            return carry2

        lax.fori_loop(0, HC // 2, chunk2, 0)
        return carry

    lax.fori_loop(0, 2, half, 0)
    plsc.subcore_barrier()

    pltpu.sync_copy(acc_sh.at[pl.ds(row0, RPT)],
                    out_hbm.at[pl.ds(c * NP + row0, RPT)])


def _degree_body(dst_hbm, ones_hbm, zer_hbm, out_hbm,
                 dst_v, ones_v, acc_sh):
    c = lax.axis_index("c")
    s = lax.axis_index("s")
    w = s * NC + c
    row0 = s * RPT

    pltpu.sync_copy(zer_hbm, acc_sh.at[pl.ds(row0, RPT)])
    pltpu.sync_copy(ones_hbm, ones_v)
    pltpu.sync_copy(dst_hbm.at[pl.ds(w * NCHUNK, NCHUNK)], dst_v)
    plsc.subcore_barrier()

    def chunk(i, carry):
        pltpu.sync_copy(ones_v, acc_sh.at[dst_v.at[i]], add=True)
        return carry

    lax.fori_loop(0, NCHUNK, chunk, 0)
    plsc.subcore_barrier()

    pltpu.sync_copy(acc_sh.at[pl.ds(row0, RPT)],
                    out_hbm.at[pl.ds(c * NP + row0, RPT)])


_seg = pl.kernel(
    _seg_body,
    out_type=[jax.ShapeDtypeStruct((NC * NP, D), jnp.float32)],
    mesh=_SC_MESH,
    scratch_types=[
        pltpu.VMEM((HC, K), jnp.int32),
        pltpu.VMEM((HC, K), jnp.int32),
        pltpu.VMEM((K, D), jnp.float32),
        pltpu.VMEM((K, D), jnp.float32),
        pltpu.VMEM_SHARED((NP, D), jnp.float32),
        pltpu.SemaphoreType.DMA,
        pltpu.SemaphoreType.DMA,
    ],
)

_degree = pl.kernel(
    _degree_body,
    out_type=[jax.ShapeDtypeStruct((NC * NP, D), jnp.float32)],
    mesh=_SC_MESH,
    scratch_types=[
        pltpu.VMEM((NCHUNK, K), jnp.int32),
        pltpu.VMEM((K, D), jnp.float32),
        pltpu.VMEM_SHARED((NP, D), jnp.float32),
    ],
)


@jax.jit
def _run(deg_idx, edge_index, emb, Wl0, bl0, Wr0, Wl1, bl1, Wr1):
    src = jnp.concatenate(
        [edge_index[0].astype(jnp.int32), jnp.zeros((EP - E,), jnp.int32)]
    ).reshape(NW * NCHUNK, K)
    pad_dst = N + jnp.arange(EP - E, dtype=jnp.int32) % (NP - N)
    dst = jnp.concatenate(
        [edge_index[1].astype(jnp.int32), pad_dst]
    ).reshape(NW * NCHUNK, K)
    di = deg_idx.reshape(N, 1).astype(jnp.int32)

    zeros_a = jnp.zeros((RPT, D), jnp.float32)
    ones_k = jnp.ones((K, D), jnp.float32)

    (dp,) = _degree(dst, ones_k, zeros_a)

    y0, z0 = _prep(di, emb, Wl0, Wr0)

    (s1p,) = _seg(y0, src, dst, zeros_a)

    y1, z1, dinv = _mid(s1p, dp, z0, bl0.reshape(1, D), Wl1, Wr1)

    (s2p,) = _seg(y1, src, dst, zeros_a)

    (pooled,) = _fin(s2p, dinv, z1, bl1.reshape(1, D))
    return pooled[None]


def kernel(deg_idx, edge_index, batch, emb, Wl0, bl0, Wr0, Wl1, bl1, Wr1):
    del batch
    return _run(deg_idx, edge_index, emb, Wl0, bl0, Wr0, Wl1, bl1, Wr1)
